# baseline (speedup 1.0000x reference)
import jax
import jax.numpy as jnp
from jax import lax
from jax.experimental import pallas as pl
from jax.experimental.pallas import tpu as pltpu

SL = 30


def _topk_body(wrow_ref, wcol_ref, out_ref):
    st = wrow_ref.shape[1]
    wi = jnp.broadcast_to(wrow_ref[...], (st, st))
    wj = jnp.broadcast_to(wcol_ref[...], (st, st))
    jj = lax.broadcasted_iota(jnp.int32, (st, st), 0)
    ii = lax.broadcasted_iota(jnp.int32, (st, st), 1)
    beats = (wj > wi) | ((wj == wi) & (jj < ii))
    rank = jnp.sum(beats.astype(jnp.int32), axis=0, keepdims=True)
    rb = jnp.broadcast_to(rank, (SL, st))
    rr = lax.broadcasted_iota(jnp.int32, (SL, st), 0)
    iidx = lax.broadcasted_iota(jnp.int32, (SL, st), 1)
    out_ref[...] = jnp.sum(jnp.where(rb == rr, iidx, 0), axis=1, keepdims=True)


def _copy_body(_, in_ref, out_ref):
    out_ref[...] = in_ref[...]


def kernel(x, small_x, thw, tem_x, tem_thw, tem_weights, tem_positions,
           tem_indices):
    h, w = 16, 16
    xdim = x.shape[-1]
    t = x.shape[0] // ((h // 2) * (w // 2) * 2 * 2)
    rows_per_frame = x.shape[0] // t
    srows = small_x.shape[0] // t
    st = tem_weights.shape[0]
    K = srows * xdim

    klarge2 = pl.pallas_call(
        _topk_body,
        out_shape=jax.ShapeDtypeStruct((SL, 1), jnp.int32),
    )(tem_weights.reshape(1, st), tem_weights.reshape(st, 1))
    klarge = klarge2.reshape(SL)

    tem3 = tem_x.reshape(st, srows, xdim)
    cent = pl.pallas_call(
        _copy_body,
        grid_spec=pltpu.PrefetchScalarGridSpec(
            num_scalar_prefetch=1,
            grid=(SL,),
            in_specs=[pl.BlockSpec((1, srows, xdim),
                                   lambda i, kl: (kl[i], 0, 0))],
            out_specs=pl.BlockSpec((1, srows, xdim), lambda i, kl: (i, 0, 0)),
        ),
        out_shape=jax.ShapeDtypeStruct((SL, srows, xdim), jnp.float32),
    )(klarge, tem3)

    centf = cent.reshape(SL, K)
    sflat = small_x.reshape(t, K)
    NK = 12
    TK = K // NK

    def _dist_body(c_ref, s_ref, o_ref, acc_ref, s2_ref):
        k = pl.program_id(0)

        @pl.when(k == 0)
        def _init():
            acc_ref[...] = jnp.zeros_like(acc_ref)
            s2_ref[...] = jnp.zeros_like(s2_ref)

        c = c_ref[...]
        s = s_ref[...]
        acc_ref[...] += lax.dot_general(
            c, s, (((1,), (1,)), ((), ())),
            preferred_element_type=jnp.float32)
        s2_ref[...] += lax.dot_general(
            jnp.ones((1, TK), jnp.float32), s * s,
            (((1,), (1,)), ((), ())),
            preferred_element_type=jnp.float32)

        @pl.when(k == NK - 1)
        def _finish():
            score = s2_ref[...] - 2.0 * acc_ref[...]
            m = jnp.min(score, axis=1, keepdims=True)
            ji = lax.broadcasted_iota(jnp.int32, (SL, t), 1)
            big = jnp.where(score == m, ji, jnp.int32(2**30))
            o_ref[...] = jnp.min(big, axis=1, keepdims=True)

    idx2 = pl.pallas_call(
        _dist_body,
        grid=(NK,),
        in_specs=[
            pl.BlockSpec((SL, TK), lambda k: (0, k)),
            pl.BlockSpec((t, TK), lambda k: (0, k)),
        ],
        out_specs=pl.BlockSpec((SL, 1), lambda k: (0, 0)),
        out_shape=jax.ShapeDtypeStruct((SL, 1), jnp.int32),
        scratch_shapes=[
            pltpu.VMEM((SL, t), jnp.float32),
            pltpu.VMEM((1, t), jnp.float32),
        ],
    )(centf, sflat)
    idx = idx2.reshape(SL)

    x3 = x.reshape(t, rows_per_frame, xdim)
    spa_x = pl.pallas_call(
        _copy_body,
        grid_spec=pltpu.PrefetchScalarGridSpec(
            num_scalar_prefetch=1,
            grid=(SL,),
            in_specs=[pl.BlockSpec((1, rows_per_frame, xdim),
                                   lambda i, ii: (ii[i], 0, 0))],
            out_specs=pl.BlockSpec((1, rows_per_frame, xdim),
                                   lambda i, ii: (i, 0, 0)),
        ),
        out_shape=jax.ShapeDtypeStruct((SL, rows_per_frame, xdim),
                                       jnp.float32),
    )(idx, x3)

    spa_thw = thw.at[0].set(SL)
    return spa_x, spa_thw, idx
